# baseline (speedup 1.0000x reference)
import jax
import jax.numpy as jnp
from jax import lax
from jax.experimental import pallas as pl
from jax.experimental.pallas import tpu as pltpu

_B = 16384
_D = 128
_NC = 8
_GAMMA2 = 0.25
_CENTER_MARGIN = 1.0
_LAMBDA_CENTER = 0.001
_EPS = 1e-06

_CH = 1024
_NCH = _B // _CH
_NBUF = 8


def _body(labels_ref, centers_ref, x_hbm, out_ref, xbuf, d2_ref, sem):
    c = centers_ref[...]
    cn = jnp.sum(c * c, axis=1, keepdims=True)
    ones = jnp.ones((1, _D), jnp.float32)

    def copy(i):
        return pltpu.make_async_copy(
            x_hbm.at[pl.ds(i * _CH, _CH), :], xbuf.at[i % _NBUF],
            sem.at[i % _NBUF])

    for i in range(_NBUF):
        copy(i).start()

    lab = labels_ref[...]
    real = lab == 0
    num_real_f = jnp.sum(jnp.where(real, 1.0, 0.0))
    num_real = num_real_f.astype(jnp.int32)
    num_forged = _B - num_real
    k_real = jnp.maximum(1, (7 * num_real + 9) // 10)
    k_forged = jnp.maximum(1, (7 * num_forged + 9) // 10)
    k_real_f = k_real.astype(jnp.float32)
    k_forged_f = k_forged.astype(jnp.float32)

    vmin = jnp.float32(jnp.inf)
    vmax = jnp.float32(-jnp.inf)
    for i in range(_NCH):
        copy(i).wait()
        x = xbuf[i % _NBUF]
        if i + _NBUF < _NCH:
            copy(i + _NBUF).start()
        cxT = lax.dot_general(c, x, (((1,), (1,)), ((), ())),
                              preferred_element_type=jnp.float32)
        xnT = lax.dot_general(ones, x * x, (((1,), (1,)), ((), ())),
                              preferred_element_type=jnp.float32)
        g = jnp.min(cn - 2.0 * cxT, axis=0, keepdims=True)
        row = jnp.maximum(g + xnT, 0.0)
        d2_ref[pl.ds(i, 1), :] = row
        vmin = jnp.minimum(vmin, jnp.min(row))
        vmax = jnp.maximum(vmax, jnp.max(row))

    d2a = d2_ref[...]
    bits = lax.bitcast_convert_type(d2a, jnp.int32)
    rbits = jnp.where(real, bits, jnp.int32(-1))
    fbits = jnp.where(real, jnp.int32(0x7FFFFFFF), bits)

    bmin = lax.bitcast_convert_type(vmin, jnp.int32)
    bmax = lax.bitcast_convert_type(vmax, jnp.int32) + 1

    def it(_, carry):
        lo_r, hi_r, lo_f, hi_f = carry
        mid_r = lo_r + (hi_r - lo_r) // 2
        mid_f = lo_f + (hi_f - lo_f) // 2
        contrib = ((rbits >= mid_r).astype(jnp.int32)
                   + ((fbits < mid_f).astype(jnp.int32) << 15))
        s = jnp.sum(contrib)
        cnt_r = s & 32767
        cnt_f = s >> 15
        ge = cnt_r >= k_real
        lo_r = jnp.where(ge, mid_r, lo_r)
        hi_r = jnp.where(ge, hi_r, mid_r)
        lt = cnt_f < k_forged
        lo_f = jnp.where(lt, mid_f, lo_f)
        hi_f = jnp.where(lt, hi_f, mid_f)
        return lo_r, hi_r, lo_f, hi_f

    lo_r, _, lo_f, _ = lax.fori_loop(0, 20, it, (bmin, bmax, bmin, bmax))

    v_r = lax.bitcast_convert_type(lo_r, jnp.float32)
    gt = rbits > lo_r
    sum_gt = jnp.sum(jnp.where(gt, d2a, 0.0))
    cnt_gt = jnp.sum(jnp.where(gt, 1.0, 0.0))
    top_sum = sum_gt + (k_real_f - cnt_gt) * v_r
    real_loss = top_sum / (2.0 * (k_real_f + _EPS))
    real_loss = jnp.where(num_real > 0, real_loss, 0.0)

    v_f = lax.bitcast_convert_type(lo_f, jnp.float32)
    ltm = fbits < lo_f
    sum_lt = jnp.sum(jnp.where(ltm, d2a, 0.0))
    cnt_lt = jnp.sum(jnp.where(ltm, 1.0, 0.0))
    bot_sum = sum_lt + (k_forged_f - cnt_lt) * v_f
    avg_forged = bot_sum / (2.0 * (k_forged_f + _EPS))
    forged_term = jnp.where(num_forged > 0,
                            jnp.minimum(avg_forged, _GAMMA2), 0.0)

    cc = lax.dot_general(c, c, (((1,), (1,)), ((), ())),
                         preferred_element_type=jnp.float32)
    cn2 = jnp.sum(c * c, axis=1)
    d2m = jnp.maximum(cn2[:, None] + cn2[None, :] - 2.0 * cc, 0.0)
    ii = lax.broadcasted_iota(jnp.int32, (_NC, _NC), 0)
    jj = lax.broadcasted_iota(jnp.int32, (_NC, _NC), 1)
    upper = jj > ii
    dist = jnp.sqrt(d2m + _EPS)
    hinge = jnp.maximum(_CENTER_MARGIN - dist, 0.0)
    num_pairs = _NC * (_NC - 1) // 2
    repulsion = _LAMBDA_CENTER * (
        jnp.sum(jnp.where(upper, hinge, 0.0)) / (num_pairs + _EPS))

    out_ref[0, 0] = real_loss - forged_term + repulsion


def kernel(cls_global, labels, centers):
    labels2d = labels.reshape(_NCH, _CH)
    out = pl.pallas_call(
        _body,
        in_specs=[
            pl.BlockSpec(memory_space=pltpu.VMEM),
            pl.BlockSpec(memory_space=pltpu.VMEM),
            pl.BlockSpec(memory_space=pl.ANY),
        ],
        out_specs=pl.BlockSpec(memory_space=pltpu.SMEM),
        out_shape=jax.ShapeDtypeStruct((1, 1), jnp.float32),
        scratch_shapes=[
            pltpu.VMEM((_NBUF, _CH, _D), jnp.float32),
            pltpu.VMEM((_NCH, _CH), jnp.float32),
            pltpu.SemaphoreType.DMA((_NBUF,)),
        ],
    )(labels2d, centers, cls_global)
    return out[0, 0]


# speedup vs baseline: 1.2517x; 1.2517x over previous
import jax
import jax.numpy as jnp
from jax import lax
from jax.experimental import pallas as pl
from jax.experimental.pallas import tpu as pltpu

_B = 16384
_D = 128
_NC = 8
_GAMMA2 = 0.25
_CENTER_MARGIN = 1.0
_LAMBDA_CENTER = 0.001
_EPS = 1e-06

_CH = 2048
_NCH = _B // _CH
_NBUF = 4


def _body(labels_ref, centers_ref, x_hbm, out_ref, xbuf, d2_ref, sem):
    c = centers_ref[...]
    cn = jnp.sum(c * c, axis=1, keepdims=True)
    ones = jnp.ones((1, _D), jnp.float32)

    def copy(i):
        return pltpu.make_async_copy(
            x_hbm.at[pl.ds(i * _CH, _CH), :], xbuf.at[i % _NBUF],
            sem.at[i % _NBUF])

    for i in range(_NBUF):
        copy(i).start()

    lab = labels_ref[...]
    real = lab == 0
    num_real_f = jnp.sum(jnp.where(real, 1.0, 0.0))
    num_real = num_real_f.astype(jnp.int32)
    num_forged = _B - num_real
    k_real = jnp.maximum(1, (7 * num_real + 9) // 10)
    k_forged = jnp.maximum(1, (7 * num_forged + 9) // 10)
    k_real_f = k_real.astype(jnp.float32)
    k_forged_f = k_forged.astype(jnp.float32)

    for i in range(_NCH):
        copy(i).wait()
        x = xbuf[i % _NBUF]
        if i + _NBUF < _NCH:
            copy(i + _NBUF).start()
        cxT = lax.dot_general(c, x, (((1,), (1,)), ((), ())),
                              preferred_element_type=jnp.float32)
        xnT = lax.dot_general(ones, x * x, (((1,), (1,)), ((), ())),
                              preferred_element_type=jnp.float32)
        g = jnp.min(cn - 2.0 * cxT, axis=0, keepdims=True)
        d2_ref[pl.ds(i, 1), :] = jnp.maximum(g + xnT, 0.0)

    d2a = d2_ref[...]
    bits = lax.bitcast_convert_type(d2a, jnp.int32)
    rbits = jnp.where(real, bits, jnp.int32(-1))
    fbits = jnp.where(real, jnp.int32(0x7FFFFFFF), bits)

    bmin = lax.bitcast_convert_type(jnp.min(d2a), jnp.int32)
    bmax = lax.bitcast_convert_type(jnp.max(d2a), jnp.int32) + 1

    def it(_, carry):
        lo_r, hi_r, lo_f, hi_f = carry
        mid_r = lo_r + (hi_r - lo_r) // 2
        mid_f = lo_f + (hi_f - lo_f) // 2
        contrib = ((rbits >= mid_r).astype(jnp.int32)
                   + ((fbits < mid_f).astype(jnp.int32) << 15))
        s = jnp.sum(contrib)
        cnt_r = s & 32767
        cnt_f = s >> 15
        ge = cnt_r >= k_real
        lo_r = jnp.where(ge, mid_r, lo_r)
        hi_r = jnp.where(ge, hi_r, mid_r)
        lt = cnt_f < k_forged
        lo_f = jnp.where(lt, mid_f, lo_f)
        hi_f = jnp.where(lt, hi_f, mid_f)
        return lo_r, hi_r, lo_f, hi_f

    lo_r, _, lo_f, _ = lax.fori_loop(0, 20, it, (bmin, bmax, bmin, bmax))

    v_r = lax.bitcast_convert_type(lo_r, jnp.float32)
    gt = rbits > lo_r
    sum_gt = jnp.sum(jnp.where(gt, d2a, 0.0))
    cnt_gt = jnp.sum(jnp.where(gt, 1.0, 0.0))
    top_sum = sum_gt + (k_real_f - cnt_gt) * v_r
    real_loss = top_sum / (2.0 * (k_real_f + _EPS))
    real_loss = jnp.where(num_real > 0, real_loss, 0.0)

    v_f = lax.bitcast_convert_type(lo_f, jnp.float32)
    ltm = fbits < lo_f
    sum_lt = jnp.sum(jnp.where(ltm, d2a, 0.0))
    cnt_lt = jnp.sum(jnp.where(ltm, 1.0, 0.0))
    bot_sum = sum_lt + (k_forged_f - cnt_lt) * v_f
    avg_forged = bot_sum / (2.0 * (k_forged_f + _EPS))
    forged_term = jnp.where(num_forged > 0,
                            jnp.minimum(avg_forged, _GAMMA2), 0.0)

    cc = lax.dot_general(c, c, (((1,), (1,)), ((), ())),
                         preferred_element_type=jnp.float32)
    cn2 = jnp.sum(c * c, axis=1)
    d2m = jnp.maximum(cn2[:, None] + cn2[None, :] - 2.0 * cc, 0.0)
    ii = lax.broadcasted_iota(jnp.int32, (_NC, _NC), 0)
    jj = lax.broadcasted_iota(jnp.int32, (_NC, _NC), 1)
    upper = jj > ii
    dist = jnp.sqrt(d2m + _EPS)
    hinge = jnp.maximum(_CENTER_MARGIN - dist, 0.0)
    num_pairs = _NC * (_NC - 1) // 2
    repulsion = _LAMBDA_CENTER * (
        jnp.sum(jnp.where(upper, hinge, 0.0)) / (num_pairs + _EPS))

    out_ref[0, 0] = real_loss - forged_term + repulsion


def kernel(cls_global, labels, centers):
    labels2d = labels.reshape(_NCH, _CH)
    out = pl.pallas_call(
        _body,
        in_specs=[
            pl.BlockSpec(memory_space=pltpu.VMEM),
            pl.BlockSpec(memory_space=pltpu.VMEM),
            pl.BlockSpec(memory_space=pl.ANY),
        ],
        out_specs=pl.BlockSpec(memory_space=pltpu.SMEM),
        out_shape=jax.ShapeDtypeStruct((1, 1), jnp.float32),
        scratch_shapes=[
            pltpu.VMEM((_NBUF, _CH, _D), jnp.float32),
            pltpu.VMEM((_NCH, _CH), jnp.float32),
            pltpu.SemaphoreType.DMA((_NBUF,)),
        ],
    )(labels2d, centers, cls_global)
    return out[0, 0]
